# baseline (speedup 1.0000x reference)
import functools

import jax
import jax.numpy as jnp
import numpy as np
from jax import lax
from jax.experimental import pallas as pl
from jax.experimental.pallas import tpu as pltpu
from jax.experimental.pallas import tpu_sc as plsc

N = 10000
D = 128
DW = D // 2
E = 320000
NC = 2
NS = 16
NW = NC * NS
CH = 80
NCHUNK = 128
EPW = NCHUNK * CH
EPAD = NW * EPW
RPOOL = 10112
RPT = RPOOL // NS
NBUF = 4
NGROUP = NCHUNK // NBUF

_PIDX = np.arange(D).reshape(4, 16, 2).transpose(0, 2, 1).reshape(D)


@functools.cache
def _make_sc_pool():
    mesh = plsc.VectorSubcoreMesh(core_axis_name="c", subcore_axis_name="s")
    return functools.partial(
        pl.kernel,
        mesh=mesh,
        out_type=jax.ShapeDtypeStruct((NC, RPOOL, D), jnp.float32),
        scratch_types=[
            pltpu.VMEM((NBUF, CH), jnp.int32),
            pltpu.VMEM((NBUF, CH), jnp.int32),
            pltpu.VMEM((NBUF, CH, DW), jnp.int32),
            pltpu.VMEM((2, CH, D), jnp.float32),
            pltpu.VMEM((2, CH), jnp.int32),
            pltpu.VMEM_SHARED((RPOOL, D), jnp.float32),
        ] + [pltpu.SemaphoreType.DMA] * (2 * NBUF + 2),
        compiler_params=pltpu.CompilerParams(needs_layout_passes=False, use_tc_tiling_on_sc=False),
    )(_sc_pool_body)


def _sc_pool_body(feat_hbm, src_hbm, dst_hbm, zeros_hbm, out_hbm,
                  src_v, dst_v, rows_v, fbuf_v, sdst_v, pool_sh, *sems):
    isems = sems[:NBUF]
    gsems = sems[NBUF:2 * NBUF]
    ssems = sems[2 * NBUF:]
    c = lax.axis_index("c")
    s = lax.axis_index("s")
    wid = c * NS + s
    pltpu.sync_copy(zeros_hbm.at[pl.ds(s * RPT, RPT)],
                    pool_sh.at[pl.ds(s * RPT, RPT)])

    def fetch_idx(t, b):
        pltpu.async_copy(src_hbm.at[wid, t], src_v.at[b], isems[b])
        pltpu.async_copy(dst_hbm.at[wid, t], dst_v.at[b], isems[b])

    def wait_idx(t, b):
        pltpu.make_async_copy(src_hbm.at[wid, t], src_v.at[b], isems[b]).wait()
        pltpu.make_async_copy(dst_hbm.at[wid, t], dst_v.at[b], isems[b]).wait()

    def gather(b):
        pltpu.async_copy(feat_hbm.at[src_v.at[b]], rows_v.at[b], gsems[b])

    def wait_gather(b):
        pltpu.make_async_copy(feat_hbm.at[src_v.at[b]], rows_v.at[b],
                              gsems[b]).wait()

    def wait_scatter(fs):
        pltpu.make_async_copy(fbuf_v.at[fs], pool_sh.at[sdst_v.at[fs]],
                              ssems[fs]).wait()

    def scatter(b, t, first_round):
        fs = b % 2
        if first_round:
            if isinstance(t, int):
                if t >= 2:
                    wait_scatter(fs)
            else:
                @pl.when(t >= 2)
                def _():
                    wait_scatter(fs)
        else:
            wait_scatter(fs)
        for k in range(CH // 16):
            sdst_v[fs, pl.ds(16 * k, 16)] = dst_v[b, pl.ds(16 * k, 16)]

        def unpack_rows(r4, carry):
            for dr in range(8):
                r = 8 * r4 + dr
                for g in range(4):
                    w = rows_v[b, r, pl.ds(16 * g, 16)]
                    lo = plsc.bitcast(w << 16, jnp.float32)
                    hi = plsc.bitcast(w, jnp.float32)
                    fbuf_v[fs, r, pl.ds(32 * g, 16)] = lo
                    fbuf_v[fs, r, pl.ds(32 * g + 16, 16)] = hi
            return carry

        lax.fori_loop(0, CH // 8, unpack_rows, 0)
        pltpu.async_copy(fbuf_v.at[fs], pool_sh.at[sdst_v.at[fs]], ssems[fs],
                         add=True)

    for b in range(NBUF):
        fetch_idx(b, b)
    for b in range(NBUF - 1):
        wait_idx(b, b)
        gather(b)
    plsc.subcore_barrier()

    def body0(g, carry):
        t0 = g * NBUF
        for b in range(NBUF):
            b1 = (b - 1) % NBUF
            wait_idx(t0 + b + NBUF - 1, b1)
            gather(b1)
            wait_gather(b)
            scatter(b, t0 + b, first_round=True)
            fetch_idx(t0 + b + NBUF, b)
        return carry

    lax.fori_loop(0, 1, body0, 0)

    def body(g, carry):
        t0 = g * NBUF
        for b in range(NBUF):
            b1 = (b - 1) % NBUF
            wait_idx(t0 + b + NBUF - 1, b1)
            gather(b1)
            wait_gather(b)
            scatter(b, t0 + b, first_round=False)
            fetch_idx(t0 + b + NBUF, b)
        return carry

    lax.fori_loop(1, NGROUP - 1, body, 0)

    wait_idx(NCHUNK - 1, NBUF - 1)
    gather(NBUF - 1)
    for b in range(NBUF):
        wait_gather(b)
        scatter(b, NCHUNK - NBUF + b, first_round=False)

    wait_scatter(0)
    wait_scatter(1)

    plsc.subcore_barrier()
    pltpu.sync_copy(pool_sh.at[pl.ds(s * RPT, RPT)],
                    out_hbm.at[c, pl.ds(s * RPT, RPT)])


def _dense_body(f_ref, pa_ref, pb_ref, wa_ref, wb_ref, b_ref, o_ref):
    f = f_ref[...]
    p = pa_ref[0] + pb_ref[0]
    acc = jnp.dot(f + p, wa_ref[...], preferred_element_type=jnp.float32,
                  precision=lax.Precision.HIGHEST)
    acc = acc + jnp.dot(f * p, wb_ref[...], preferred_element_type=jnp.float32,
                        precision=lax.Precision.HIGHEST)
    o_ref[...] = jnp.maximum(acc + b_ref[...], 0.0)


def _dense_norm_body(f_ref, pa_ref, pb_ref, wa_ref, wb_ref, b_ref, o_ref):
    f = f_ref[...]
    p = pa_ref[0] + pb_ref[0]
    acc = jnp.dot(f + p, wa_ref[...], preferred_element_type=jnp.float32,
                  precision=lax.Precision.HIGHEST)
    acc = acc + jnp.dot(f * p, wb_ref[...], preferred_element_type=jnp.float32,
                        precision=lax.Precision.HIGHEST)
    h = jnp.maximum(acc + b_ref[...], 0.0)
    nrm = jnp.sqrt(jnp.sum(h * h, axis=-1, keepdims=True))
    o_ref[...] = h / jnp.maximum(nrm, 1e-12)


def _dense(fperm, parts, wa, wb, brow, normalize):
    body = _dense_norm_body if normalize else _dense_body
    return pl.pallas_call(
        body,
        grid=(1,),
        out_shape=jax.ShapeDtypeStruct((N, D), jnp.float32),
        in_specs=[
            pl.BlockSpec((N, D), lambda i: (0, 0)),
            pl.BlockSpec((1, N, D), lambda i: (0, 0, 0)),
            pl.BlockSpec((1, N, D), lambda i: (1, 0, 0)),
            pl.BlockSpec((D, D), lambda i: (0, 0)),
            pl.BlockSpec((D, D), lambda i: (0, 0)),
            pl.BlockSpec((1, D), lambda i: (0, 0)),
        ],
        out_specs=pl.BlockSpec((N, D), lambda i: (0, 0)),
    )(fperm, parts, parts, wa, wb, brow)


def _pack(f):
    return lax.bitcast_convert_type(
        f.astype(jnp.bfloat16).reshape(N, DW, 2), jnp.int32)


def kernel(x, edge_index, W1, b1, W2, b2):
    src = edge_index[0]
    dst = edge_index[1]
    pad = EPAD - E
    srcp = jnp.concatenate([src, jnp.zeros((pad,), jnp.int32)])
    srcp = srcp.reshape(NW, NCHUNK, CH)
    dstp = jnp.concatenate([dst, jnp.full((pad,), N, jnp.int32)])
    dstp = dstp.reshape(NW, NCHUNK, CH)
    zeros = jnp.zeros((RPOOL, D), jnp.float32)

    pidx = jnp.asarray(_PIDX)
    w1a = W1[:, :D].T[pidx, :]
    w1b = W1[:, D:].T[pidx, :]
    w2a = W2[:, :D].T[pidx, :]
    w2b = W2[:, D:].T[pidx, :]

    sc_pool = _make_sc_pool()
    parts1 = sc_pool(_pack(x), srcp, dstp, zeros)
    h1 = _dense(x[:, pidx], parts1, w1a, w1b, b1.reshape(1, D),
                normalize=False)
    parts2 = sc_pool(_pack(h1), srcp, dstp, zeros)
    return _dense(h1[:, pidx], parts2, w2a, w2b, b2.reshape(1, D),
                  normalize=True)


# speedup vs baseline: 1.0194x; 1.0194x over previous
import functools

import jax
import jax.numpy as jnp
import numpy as np
from jax import lax
from jax.experimental import pallas as pl
from jax.experimental.pallas import tpu as pltpu
from jax.experimental.pallas import tpu_sc as plsc

N = 10000
D = 128
DW = D // 2
E = 320000
NC = 2
NS = 16
NW = NC * NS
CH = 128
NCHUNK = 80
EPW = NCHUNK * CH
EPAD = NW * EPW
RPOOL = 10112
RPT = RPOOL // NS
NBUF = 2
NGROUP = NCHUNK // NBUF

_PIDX = np.arange(D).reshape(4, 16, 2).transpose(0, 2, 1).reshape(D)


@functools.cache
def _make_sc_pool():
    mesh = plsc.VectorSubcoreMesh(core_axis_name="c", subcore_axis_name="s")
    return functools.partial(
        pl.kernel,
        mesh=mesh,
        out_type=jax.ShapeDtypeStruct((NC, RPOOL, D), jnp.float32),
        scratch_types=[
            pltpu.VMEM((NBUF, CH), jnp.int32),
            pltpu.VMEM((NBUF, CH), jnp.int32),
            pltpu.VMEM((NBUF, CH, DW), jnp.int32),
            pltpu.VMEM((2, CH, D), jnp.float32),
            pltpu.VMEM((2, CH), jnp.int32),
            pltpu.VMEM_SHARED((RPOOL, D), jnp.float32),
        ] + [pltpu.SemaphoreType.DMA] * (2 * NBUF + 2),
        compiler_params=pltpu.CompilerParams(needs_layout_passes=False, use_tc_tiling_on_sc=False),
    )(_sc_pool_body)


def _sc_pool_body(feat_hbm, src_hbm, dst_hbm, zeros_hbm, out_hbm,
                  src_v, dst_v, rows_v, fbuf_v, sdst_v, pool_sh, *sems):
    isems = sems[:NBUF]
    gsems = sems[NBUF:2 * NBUF]
    ssems = sems[2 * NBUF:]
    c = lax.axis_index("c")
    s = lax.axis_index("s")
    wid = c * NS + s
    pltpu.sync_copy(zeros_hbm.at[pl.ds(s * RPT, RPT)],
                    pool_sh.at[pl.ds(s * RPT, RPT)])

    def fetch_idx(t, b):
        pltpu.async_copy(src_hbm.at[wid, t], src_v.at[b], isems[b])
        pltpu.async_copy(dst_hbm.at[wid, t], dst_v.at[b], isems[b])

    def wait_idx(t, b):
        pltpu.make_async_copy(src_hbm.at[wid, t], src_v.at[b], isems[b]).wait()
        pltpu.make_async_copy(dst_hbm.at[wid, t], dst_v.at[b], isems[b]).wait()

    def gather(b):
        pltpu.async_copy(feat_hbm.at[src_v.at[b]], rows_v.at[b], gsems[b])

    def wait_gather(b):
        pltpu.make_async_copy(feat_hbm.at[src_v.at[b]], rows_v.at[b],
                              gsems[b]).wait()

    def wait_scatter(fs):
        pltpu.make_async_copy(fbuf_v.at[fs], pool_sh.at[sdst_v.at[fs]],
                              ssems[fs]).wait()

    def scatter(b, t, first_round):
        fs = b % 2
        if first_round:
            if isinstance(t, int):
                if t >= 2:
                    wait_scatter(fs)
            else:
                @pl.when(t >= 2)
                def _():
                    wait_scatter(fs)
        else:
            wait_scatter(fs)
        for k in range(CH // 16):
            sdst_v[fs, pl.ds(16 * k, 16)] = dst_v[b, pl.ds(16 * k, 16)]

        def unpack_rows(r4, carry):
            for dr in range(4):
                r = 4 * r4 + dr
                for g in range(4):
                    w = rows_v[b, r, pl.ds(16 * g, 16)]
                    lo = plsc.bitcast(w << 16, jnp.float32)
                    hi = plsc.bitcast(w, jnp.float32)
                    fbuf_v[fs, r, pl.ds(32 * g, 16)] = lo
                    fbuf_v[fs, r, pl.ds(32 * g + 16, 16)] = hi
            return carry

        lax.fori_loop(0, CH // 4, unpack_rows, 0)
        pltpu.async_copy(fbuf_v.at[fs], pool_sh.at[sdst_v.at[fs]], ssems[fs],
                         add=True)

    for b in range(NBUF):
        fetch_idx(b, b)
    for b in range(NBUF - 1):
        wait_idx(b, b)
        gather(b)
    plsc.subcore_barrier()

    def body0(g, carry):
        t0 = g * NBUF
        for b in range(NBUF):
            b1 = (b - 1) % NBUF
            wait_idx(t0 + b + NBUF - 1, b1)
            gather(b1)
            wait_gather(b)
            scatter(b, t0 + b, first_round=True)
            fetch_idx(t0 + b + NBUF, b)
        return carry

    lax.fori_loop(0, 1, body0, 0)

    def body(g, carry):
        t0 = g * NBUF
        for b in range(NBUF):
            b1 = (b - 1) % NBUF
            wait_idx(t0 + b + NBUF - 1, b1)
            gather(b1)
            wait_gather(b)
            scatter(b, t0 + b, first_round=False)
            fetch_idx(t0 + b + NBUF, b)
        return carry

    lax.fori_loop(1, NGROUP - 1, body, 0)

    wait_idx(NCHUNK - 1, NBUF - 1)
    gather(NBUF - 1)
    for b in range(NBUF):
        wait_gather(b)
        scatter(b, NCHUNK - NBUF + b, first_round=False)

    wait_scatter(0)
    wait_scatter(1)

    plsc.subcore_barrier()
    pltpu.sync_copy(pool_sh.at[pl.ds(s * RPT, RPT)],
                    out_hbm.at[c, pl.ds(s * RPT, RPT)])


def _dense_body(f_ref, pa_ref, pb_ref, wa_ref, wb_ref, b_ref, o_ref):
    f = f_ref[...]
    p = pa_ref[0] + pb_ref[0]
    acc = jnp.dot(f + p, wa_ref[...], preferred_element_type=jnp.float32,
                  precision=lax.Precision.HIGHEST)
    acc = acc + jnp.dot(f * p, wb_ref[...], preferred_element_type=jnp.float32,
                        precision=lax.Precision.HIGHEST)
    o_ref[...] = jnp.maximum(acc + b_ref[...], 0.0)


def _dense_norm_body(f_ref, pa_ref, pb_ref, wa_ref, wb_ref, b_ref, o_ref):
    f = f_ref[...]
    p = pa_ref[0] + pb_ref[0]
    acc = jnp.dot(f + p, wa_ref[...], preferred_element_type=jnp.float32,
                  precision=lax.Precision.HIGHEST)
    acc = acc + jnp.dot(f * p, wb_ref[...], preferred_element_type=jnp.float32,
                        precision=lax.Precision.HIGHEST)
    h = jnp.maximum(acc + b_ref[...], 0.0)
    nrm = jnp.sqrt(jnp.sum(h * h, axis=-1, keepdims=True))
    o_ref[...] = h / jnp.maximum(nrm, 1e-12)


def _dense(fperm, parts, wa, wb, brow, normalize):
    body = _dense_norm_body if normalize else _dense_body
    return pl.pallas_call(
        body,
        grid=(1,),
        out_shape=jax.ShapeDtypeStruct((N, D), jnp.float32),
        in_specs=[
            pl.BlockSpec((N, D), lambda i: (0, 0)),
            pl.BlockSpec((1, N, D), lambda i: (0, 0, 0)),
            pl.BlockSpec((1, N, D), lambda i: (1, 0, 0)),
            pl.BlockSpec((D, D), lambda i: (0, 0)),
            pl.BlockSpec((D, D), lambda i: (0, 0)),
            pl.BlockSpec((1, D), lambda i: (0, 0)),
        ],
        out_specs=pl.BlockSpec((N, D), lambda i: (0, 0)),
    )(fperm, parts, parts, wa, wb, brow)


def _pack(f):
    return lax.bitcast_convert_type(
        f.astype(jnp.bfloat16).reshape(N, DW, 2), jnp.int32)


def kernel(x, edge_index, W1, b1, W2, b2):
    src = edge_index[0]
    dst = edge_index[1]
    pad = EPAD - E
    srcp = jnp.concatenate([src, jnp.zeros((pad,), jnp.int32)])
    srcp = srcp.reshape(NW, NCHUNK, CH)
    dstp = jnp.concatenate([dst, jnp.full((pad,), N, jnp.int32)])
    dstp = dstp.reshape(NW, NCHUNK, CH)
    zeros = jnp.zeros((RPOOL, D), jnp.float32)

    pidx = jnp.asarray(_PIDX)
    w1a = W1[:, :D].T[pidx, :]
    w1b = W1[:, D:].T[pidx, :]
    w2a = W2[:, :D].T[pidx, :]
    w2b = W2[:, D:].T[pidx, :]

    sc_pool = _make_sc_pool()
    parts1 = sc_pool(_pack(x), srcp, dstp, zeros)
    h1 = _dense(x[:, pidx], parts1, w1a, w1b, b1.reshape(1, D),
                normalize=False)
    parts2 = sc_pool(_pack(h1), srcp, dstp, zeros)
    return _dense(h1[:, pidx], parts2, w2a, w2b, b2.reshape(1, D),
                  normalize=True)


# speedup vs baseline: 1.0234x; 1.0039x over previous
import functools

import jax
import jax.numpy as jnp
import numpy as np
from jax import lax
from jax.experimental import pallas as pl
from jax.experimental.pallas import tpu as pltpu
from jax.experimental.pallas import tpu_sc as plsc

N = 10000
D = 128
DW = D // 2
E = 320000
NC = 2
NS = 16
NW = NC * NS
CH = 80
NCHUNK = 128
EPW = NCHUNK * CH
EPAD = NW * EPW
RPOOL = 10112
RPT = RPOOL // NS
NBUF = 4
NGROUP = NCHUNK // NBUF

_PIDX = np.arange(D).reshape(4, 16, 2).transpose(0, 2, 1).reshape(D)


@functools.cache
def _make_sc_pool():
    mesh = plsc.VectorSubcoreMesh(core_axis_name="c", subcore_axis_name="s")
    return functools.partial(
        pl.kernel,
        mesh=mesh,
        out_type=jax.ShapeDtypeStruct((NC, RPOOL, D), jnp.float32),
        scratch_types=[
            pltpu.VMEM((NBUF, CH), jnp.int32),
            pltpu.VMEM((NBUF, CH), jnp.int32),
            pltpu.VMEM((NBUF, CH, DW), jnp.int32),
            pltpu.VMEM((2, CH, D), jnp.float32),
            pltpu.VMEM((2, CH), jnp.int32),
            pltpu.VMEM_SHARED((RPOOL, D), jnp.float32),
        ] + [pltpu.SemaphoreType.DMA] * (2 * NBUF + 2),
        compiler_params=pltpu.CompilerParams(needs_layout_passes=False, use_tc_tiling_on_sc=False),
    )(_sc_pool_body)


def _sc_pool_body(feat_hbm, src_hbm, dst_hbm, zeros_hbm, out_hbm,
                  src_v, dst_v, rows_v, fbuf_v, sdst_v, pool_sh, *sems):
    isems = sems[:NBUF]
    gsems = sems[NBUF:2 * NBUF]
    ssems = sems[2 * NBUF:]
    c = lax.axis_index("c")
    s = lax.axis_index("s")
    wid = c * NS + s
    pltpu.sync_copy(zeros_hbm.at[pl.ds(s * RPT, RPT)],
                    pool_sh.at[pl.ds(s * RPT, RPT)])

    def fetch_idx(t, b):
        pltpu.async_copy(src_hbm.at[wid, t], src_v.at[b], isems[b])
        pltpu.async_copy(dst_hbm.at[wid, t], dst_v.at[b], isems[b])

    def wait_idx(t, b):
        pltpu.make_async_copy(src_hbm.at[wid, t], src_v.at[b], isems[b]).wait()
        pltpu.make_async_copy(dst_hbm.at[wid, t], dst_v.at[b], isems[b]).wait()

    def gather(b):
        pltpu.async_copy(feat_hbm.at[src_v.at[b]], rows_v.at[b], gsems[b])

    def wait_gather(b):
        pltpu.make_async_copy(feat_hbm.at[src_v.at[b]], rows_v.at[b],
                              gsems[b]).wait()

    def wait_scatter(fs):
        pltpu.make_async_copy(fbuf_v.at[fs], pool_sh.at[sdst_v.at[fs]],
                              ssems[fs]).wait()

    def scatter(b, t, first_round):
        fs = b % 2
        if first_round:
            if isinstance(t, int):
                if t >= 2:
                    wait_scatter(fs)
            else:
                @pl.when(t >= 2)
                def _():
                    wait_scatter(fs)
        else:
            wait_scatter(fs)
        for k in range(CH // 16):
            sdst_v[fs, pl.ds(16 * k, 16)] = dst_v[b, pl.ds(16 * k, 16)]

        def unpack_rows(r4, carry):
            for dr in range(4):
                r = 4 * r4 + dr
                for g in range(4):
                    w = rows_v[b, r, pl.ds(16 * g, 16)]
                    lo = plsc.bitcast(w << 16, jnp.float32)
                    hi = plsc.bitcast(w, jnp.float32)
                    fbuf_v[fs, r, pl.ds(32 * g, 16)] = lo
                    fbuf_v[fs, r, pl.ds(32 * g + 16, 16)] = hi
            return carry

        lax.fori_loop(0, CH // 4, unpack_rows, 0)
        pltpu.async_copy(fbuf_v.at[fs], pool_sh.at[sdst_v.at[fs]], ssems[fs],
                         add=True)

    for b in range(NBUF):
        fetch_idx(b, b)
    for b in range(NBUF - 1):
        wait_idx(b, b)
        gather(b)
    plsc.subcore_barrier()

    def body0(g, carry):
        t0 = g * NBUF
        for b in range(NBUF):
            b1 = (b - 1) % NBUF
            wait_idx(t0 + b + NBUF - 1, b1)
            gather(b1)
            wait_gather(b)
            scatter(b, t0 + b, first_round=True)
            fetch_idx(t0 + b + NBUF, b)
        return carry

    lax.fori_loop(0, 1, body0, 0)

    def body(g, carry):
        t0 = g * NBUF
        for b in range(NBUF):
            b1 = (b - 1) % NBUF
            wait_idx(t0 + b + NBUF - 1, b1)
            gather(b1)
            wait_gather(b)
            scatter(b, t0 + b, first_round=False)
            fetch_idx(t0 + b + NBUF, b)
        return carry

    lax.fori_loop(1, NGROUP - 1, body, 0)

    wait_idx(NCHUNK - 1, NBUF - 1)
    gather(NBUF - 1)
    for b in range(NBUF):
        wait_gather(b)
        scatter(b, NCHUNK - NBUF + b, first_round=False)

    wait_scatter(0)
    wait_scatter(1)

    plsc.subcore_barrier()
    pltpu.sync_copy(pool_sh.at[pl.ds(s * RPT, RPT)],
                    out_hbm.at[c, pl.ds(s * RPT, RPT)])


def _dense_body(f_ref, pa_ref, pb_ref, wa_ref, wb_ref, b_ref, o_ref):
    f = f_ref[...]
    p = pa_ref[0] + pb_ref[0]
    acc = jnp.dot(f + p, wa_ref[...], preferred_element_type=jnp.float32,
                  precision=lax.Precision.HIGHEST)
    acc = acc + jnp.dot(f * p, wb_ref[...], preferred_element_type=jnp.float32,
                        precision=lax.Precision.HIGHEST)
    o_ref[...] = jnp.maximum(acc + b_ref[...], 0.0)


def _dense_norm_body(f_ref, pa_ref, pb_ref, wa_ref, wb_ref, b_ref, o_ref):
    f = f_ref[...]
    p = pa_ref[0] + pb_ref[0]
    acc = jnp.dot(f + p, wa_ref[...], preferred_element_type=jnp.float32,
                  precision=lax.Precision.HIGHEST)
    acc = acc + jnp.dot(f * p, wb_ref[...], preferred_element_type=jnp.float32,
                        precision=lax.Precision.HIGHEST)
    h = jnp.maximum(acc + b_ref[...], 0.0)
    nrm = jnp.sqrt(jnp.sum(h * h, axis=-1, keepdims=True))
    o_ref[...] = h / jnp.maximum(nrm, 1e-12)


def _dense(fperm, parts, wa, wb, brow, normalize):
    body = _dense_norm_body if normalize else _dense_body
    return pl.pallas_call(
        body,
        grid=(1,),
        out_shape=jax.ShapeDtypeStruct((N, D), jnp.float32),
        in_specs=[
            pl.BlockSpec((N, D), lambda i: (0, 0)),
            pl.BlockSpec((1, N, D), lambda i: (0, 0, 0)),
            pl.BlockSpec((1, N, D), lambda i: (1, 0, 0)),
            pl.BlockSpec((D, D), lambda i: (0, 0)),
            pl.BlockSpec((D, D), lambda i: (0, 0)),
            pl.BlockSpec((1, D), lambda i: (0, 0)),
        ],
        out_specs=pl.BlockSpec((N, D), lambda i: (0, 0)),
    )(fperm, parts, parts, wa, wb, brow)


def _pack(f):
    return lax.bitcast_convert_type(
        f.astype(jnp.bfloat16).reshape(N, DW, 2), jnp.int32)


def kernel(x, edge_index, W1, b1, W2, b2):
    src = edge_index[0]
    dst = edge_index[1]
    pad = EPAD - E
    srcp = jnp.concatenate([src, jnp.zeros((pad,), jnp.int32)])
    srcp = srcp.reshape(NW, NCHUNK, CH)
    dstp = jnp.concatenate([dst, jnp.full((pad,), N, jnp.int32)])
    dstp = dstp.reshape(NW, NCHUNK, CH)
    zeros = jnp.zeros((RPOOL, D), jnp.float32)

    pidx = jnp.asarray(_PIDX)
    w1a = W1[:, :D].T[pidx, :]
    w1b = W1[:, D:].T[pidx, :]
    w2a = W2[:, :D].T[pidx, :]
    w2b = W2[:, D:].T[pidx, :]

    sc_pool = _make_sc_pool()
    parts1 = sc_pool(_pack(x), srcp, dstp, zeros)
    h1 = _dense(x[:, pidx], parts1, w1a, w1b, b1.reshape(1, D),
                normalize=False)
    parts2 = sc_pool(_pack(h1), srcp, dstp, zeros)
    return _dense(h1[:, pidx], parts2, w2a, w2b, b2.reshape(1, D),
                  normalize=True)
